# baseline (speedup 1.0000x reference)
import functools

import jax
import jax.numpy as jnp
from jax import lax
from jax.experimental import pallas as pl
from jax.experimental.pallas import tpu as pltpu
from jax.experimental.pallas import tpu_sc as plsc


def _make_gather(V, D, B):
    info = plsc.get_sparse_core_info()
    NC, NS = info.num_cores, info.num_subcores
    NW = NC * NS
    assert B % (8 * NW) == 0
    b_per_w = B // NW
    mesh = plsc.VectorSubcoreMesh(core_axis_name="c", subcore_axis_name="s")

    @functools.partial(
        pl.kernel,
        mesh=mesh,
        out_type=jax.ShapeDtypeStruct((B, D), jnp.float32),
        scratch_types=[
            pltpu.VMEM((b_per_w,), jnp.int32),
            pltpu.VMEM((b_per_w, D), jnp.float32),
            pltpu.SemaphoreType.DMA,
        ],
        compiler_params=pltpu.CompilerParams(use_tc_tiling_on_sc=False),
    )
    def gather_kernel(table_hbm, idx_hbm, out_hbm, idx_v, rows_v, sem):
        wid = lax.axis_index("s") * NC + lax.axis_index("c")
        base = wid * b_per_w
        pltpu.sync_copy(idx_hbm.at[pl.ds(base, b_per_w)], idx_v)
        pltpu.async_copy(table_hbm.at[idx_v], rows_v, sem).wait()
        pltpu.sync_copy(rows_v, out_hbm.at[pl.ds(base, b_per_w)])

    return gather_kernel


def kernel(x, table):
    B, = x.shape
    V, D = table.shape
    return _make_gather(V, D, B)(table, x.astype(jnp.int32))


# speedup vs baseline: 1.4604x; 1.4604x over previous
import functools

import jax
import jax.numpy as jnp
from jax import lax
from jax.experimental import pallas as pl
from jax.experimental.pallas import tpu as pltpu
from jax.experimental.pallas import tpu_sc as plsc


def _make_gather(V, D, B):
    info = plsc.get_sparse_core_info()
    NC, NS, L = info.num_cores, info.num_subcores, info.num_lanes
    NW = NC * NS
    assert B % (8 * NW) == 0 and B % (L * NW) == 0
    b_per_w = B // NW
    n_groups = b_per_w // L
    mesh = plsc.VectorSubcoreMesh(core_axis_name="c", subcore_axis_name="s")

    @functools.partial(
        pl.kernel,
        mesh=mesh,
        out_type=jax.ShapeDtypeStruct((B, D), jnp.float32),
        scratch_types=[
            pltpu.VMEM((b_per_w,), jnp.int32),
            pltpu.VMEM((b_per_w, D), jnp.float32),
            pltpu.SemaphoreType.DMA,
        ],
        compiler_params=pltpu.CompilerParams(needs_layout_passes=False),
    )
    def gather_kernel(table_hbm, idx_hbm, out_hbm, idx_v, rows_v, sem):
        wid = lax.axis_index("s") * NC + lax.axis_index("c")
        base = wid * b_per_w

        pltpu.sync_copy(idx_hbm.at[pl.ds(base, b_per_w)], idx_v)

        lane = lax.broadcasted_iota(jnp.int32, (L,), 0)

        def fire(g, carry):
            vec = idx_v[pl.ds(g * L, L)]
            for j in range(L):
                r = jnp.sum(jnp.where(lane == j, vec, 0))
                pltpu.make_async_copy(
                    table_hbm.at[r], rows_v.at[g * L + j], sem
                ).start()
            return carry

        lax.fori_loop(0, n_groups, fire, 0)

        def drain(i, carry):
            pltpu.make_async_copy(table_hbm.at[0], rows_v.at[i], sem).wait()
            return carry

        lax.fori_loop(0, b_per_w, drain, 0)

        pltpu.sync_copy(rows_v, out_hbm.at[pl.ds(base, b_per_w)])

    return gather_kernel


def kernel(x, table):
    B, = x.shape
    V, D = table.shape
    return _make_gather(V, D, B)(table, x.astype(jnp.int32))


# speedup vs baseline: 1.9869x; 1.3605x over previous
import functools

import jax
import jax.numpy as jnp
from jax import lax
from jax.experimental import pallas as pl
from jax.experimental.pallas import tpu as pltpu
from jax.experimental.pallas import tpu_sc as plsc


def _make_gather_t(V, D, B):
    info = plsc.get_sparse_core_info()
    NC, NS, L = info.num_cores, info.num_subcores, info.num_lanes
    NW = NC * NS
    assert D % NW == 0 and B % L == 0
    c_per_w = D // NW
    CHUNK = 8192
    assert B % CHUNK == 0
    n_chunks = B // CHUNK
    mesh = plsc.VectorSubcoreMesh(core_axis_name="c", subcore_axis_name="s")

    @functools.partial(
        pl.kernel,
        mesh=mesh,
        out_type=jax.ShapeDtypeStruct((D, B), jnp.float32),
        scratch_types=[
            pltpu.VMEM((B,), jnp.int32),
            pltpu.VMEM((V,), jnp.float32),
            pltpu.VMEM((CHUNK,), jnp.float32),
        ],
        compiler_params=pltpu.CompilerParams(needs_layout_passes=False),
    )
    def gather_kernel(tt_hbm, idx_hbm, out_hbm, idx_v, row_v, res_v):
        wid = lax.axis_index("s") * NC + lax.axis_index("c")

        pltpu.sync_copy(idx_hbm, idx_v)

        for cc in range(c_per_w):
            c = wid * c_per_w + cc
            pltpu.sync_copy(tt_hbm.at[c], row_v)

            for k in range(n_chunks):

                def gather_chunk(g, carry):
                    idxv = idx_v[pl.ds(k * CHUNK + g * L, L)]
                    res_v[pl.ds(g * L, L)] = plsc.load_gather(row_v, [idxv])
                    return carry

                lax.fori_loop(0, CHUNK // L, gather_chunk, 0)
                pltpu.sync_copy(res_v, out_hbm.at[c, pl.ds(k * CHUNK, CHUNK)])

    return gather_kernel


def kernel(x, table):
    B, = x.shape
    V, D = table.shape
    tt = jnp.swapaxes(table, 0, 1)
    out_t = _make_gather_t(V, D, B)(tt, x.astype(jnp.int32))
    return jnp.swapaxes(out_t, 0, 1)


# speedup vs baseline: 2.3555x; 1.1855x over previous
import functools

import jax
import jax.numpy as jnp
from jax import lax
from jax.experimental import pallas as pl
from jax.experimental.pallas import tpu as pltpu
from jax.experimental.pallas import tpu_sc as plsc


def _make_gather_t(V, D, B):
    info = plsc.get_sparse_core_info()
    NC, NS, L = info.num_cores, info.num_subcores, info.num_lanes
    NW = NC * NS
    assert D % NW == 0 and B % L == 0
    c_per_w = D // NW
    CHUNK = 4096
    UNROLL = 8
    assert B % CHUNK == 0 and CHUNK % (L * UNROLL) == 0
    n_chunks = B // CHUNK
    mesh = plsc.VectorSubcoreMesh(core_axis_name="c", subcore_axis_name="s")

    @functools.partial(
        pl.kernel,
        mesh=mesh,
        out_type=jax.ShapeDtypeStruct((D, B), jnp.float32),
        scratch_types=[
            pltpu.VMEM((B,), jnp.int32),
            pltpu.VMEM((V,), jnp.float32),
            pltpu.VMEM((CHUNK,), jnp.float32),
            pltpu.VMEM((CHUNK,), jnp.float32),
            pltpu.SemaphoreType.DMA,
            pltpu.SemaphoreType.DMA,
            pltpu.SemaphoreType.DMA,
            pltpu.SemaphoreType.DMA,
        ],
        compiler_params=pltpu.CompilerParams(needs_layout_passes=False),
    )
    def gather_kernel(tt_hbm, idx_hbm, out_hbm, idx_v, row_v, res0_v, res1_v,
                      sem_i, sem_r, sem_w0, sem_w1):
        wid = lax.axis_index("s") * NC + lax.axis_index("c")
        res_bufs = (res0_v, res1_v)
        sem_w = (sem_w0, sem_w1)

        cp_idx = pltpu.async_copy(idx_hbm, idx_v, sem_i)
        cp_row = pltpu.async_copy(tt_hbm.at[wid * c_per_w], row_v, sem_r)
        cp_idx.wait()
        cp_row.wait()

        pending = {0: None, 1: None}
        for cc in range(c_per_w):
            c = wid * c_per_w + cc
            if cc > 0:
                pltpu.sync_copy(tt_hbm.at[c], row_v)

            for k in range(n_chunks):
                par = k % 2
                res_v = res_bufs[par]
                if pending[par] is not None:
                    pending[par].wait()

                def gather_chunk(g, carry, k=k, res_v=res_v):
                    base = k * CHUNK + g * (L * UNROLL)
                    for u in range(UNROLL):
                        idxv = idx_v[pl.ds(base + u * L, L)]
                        res_v[pl.ds(g * (L * UNROLL) + u * L, L)] = (
                            plsc.load_gather(row_v, [idxv])
                        )
                    return carry

                lax.fori_loop(0, CHUNK // (L * UNROLL), gather_chunk, 0)
                pending[par] = pltpu.async_copy(
                    res_v, out_hbm.at[c, pl.ds(k * CHUNK, CHUNK)], sem_w[par]
                )
        for cp in pending.values():
            if cp is not None:
                cp.wait()

    return gather_kernel


def kernel(x, table):
    B, = x.shape
    V, D = table.shape
    tt = jnp.swapaxes(table, 0, 1)
    out_t = _make_gather_t(V, D, B)(tt, x.astype(jnp.int32))
    return jnp.swapaxes(out_t, 0, 1)


# speedup vs baseline: 2.7680x; 1.1751x over previous
import functools

import jax
import jax.numpy as jnp
from jax import lax
from jax.experimental import pallas as pl
from jax.experimental.pallas import tpu as pltpu
from jax.experimental.pallas import tpu_sc as plsc


def _make_gather_t(V, D, B):
    info = plsc.get_sparse_core_info()
    NC, NS, L = info.num_cores, info.num_subcores, info.num_lanes
    NW = NC * NS
    assert D % NW == 0 and B % L == 0
    c_per_w = D // NW
    CHUNK = 4096
    UNROLL = 8
    assert B % CHUNK == 0 and CHUNK % (L * UNROLL) == 0
    n_chunks = B // CHUNK
    mesh = plsc.VectorSubcoreMesh(core_axis_name="c", subcore_axis_name="s")

    @functools.partial(
        pl.kernel,
        mesh=mesh,
        out_type=jax.ShapeDtypeStruct((D, B), jnp.float32),
        scratch_types=[
            pltpu.VMEM((B,), jnp.int32),
            pltpu.VMEM((V,), jnp.float32),
            pltpu.VMEM((CHUNK,), jnp.float32),
            pltpu.VMEM((CHUNK,), jnp.float32),
            pltpu.SemaphoreType.DMA,
            pltpu.SemaphoreType.DMA,
            pltpu.SemaphoreType.DMA,
            pltpu.SemaphoreType.DMA,
        ],
        compiler_params=pltpu.CompilerParams(needs_layout_passes=False),
    )
    def gather_kernel(tt_hbm, idx_hbm, out_hbm, idx_v, row_v, res0_v, res1_v,
                      sem_i, sem_r, sem_w0, sem_w1):
        wid = lax.axis_index("s") * NC + lax.axis_index("c")
        res_bufs = (res0_v, res1_v)
        sem_w = (sem_w0, sem_w1)

        cp_idx = pltpu.async_copy(idx_hbm, idx_v, sem_i)
        cp_row = pltpu.async_copy(tt_hbm.at[wid * c_per_w], row_v, sem_r)
        cp_idx.wait()
        cp_row.wait()

        pending = {0: None, 1: None}
        for cc in range(c_per_w):
            c = wid * c_per_w + cc
            if cc > 0:
                pltpu.sync_copy(tt_hbm.at[c], row_v)

            for k in range(n_chunks):
                par = k % 2
                res_v = res_bufs[par]
                if pending[par] is not None:
                    pending[par].wait()

                @plsc.parallel_loop(0, CHUNK // L, unroll=UNROLL)
                def gather_chunk(g, k=k, res_v=res_v):
                    idxv = idx_v[pl.ds(k * CHUNK + g * L, L)]
                    res_v[pl.ds(g * L, L)] = plsc.load_gather(row_v, [idxv])
                pending[par] = pltpu.async_copy(
                    res_v, out_hbm.at[c, pl.ds(k * CHUNK, CHUNK)], sem_w[par]
                )
        for cp in pending.values():
            if cp is not None:
                cp.wait()

    return gather_kernel


def kernel(x, table):
    B, = x.shape
    V, D = table.shape
    tt = jnp.swapaxes(table, 0, 1)
    out_t = _make_gather_t(V, D, B)(tt, x.astype(jnp.int32))
    return jnp.swapaxes(out_t, 0, 1)
